# baseline (speedup 1.0000x reference)
import functools

import jax
import jax.numpy as jnp
from jax import lax
from jax.experimental import pallas as pl
from jax.experimental.pallas import tpu as pltpu
from jax.experimental.pallas import tpu_sc as plsc

B = 4096
L = 50
D = 128
LANES = 16
NVREG = D // LANES


@functools.cache
def _build():
    info = plsc.get_sparse_core_info()
    nw = info.num_cores * info.num_subcores
    b_per_w = B // nw
    mesh = plsc.VectorSubcoreMesh(core_axis_name="c", subcore_axis_name="s")

    @functools.partial(
        pl.kernel,
        mesh=mesh,
        compiler_params=pltpu.CompilerParams(needs_layout_passes=False),
        out_type=jax.ShapeDtypeStruct((B, D), jnp.float32),
        scratch_types=[
            pltpu.VMEM((b_per_w * L,), jnp.int32),
            pltpu.VMEM((L, b_per_w), jnp.int32),
            pltpu.VMEM((2, b_per_w, D), jnp.float32),
            [pltpu.SemaphoreType.DMA, pltpu.SemaphoreType.DMA],
        ],
    )
    def k(ids_hbm, table_hbm, out_hbm, idx_v, idxt_v, acc_v, sems):
        cid = lax.axis_index("c")
        sid = lax.axis_index("s")
        wid = sid * info.num_cores + cid
        base_row = wid * b_per_w

        pltpu.sync_copy(ids_hbm.at[pl.ds(base_row * L, b_per_w * L)], idx_v)

        @pl.loop(0, L)
        def _transpose(l):
            for g in range(b_per_w // LANES):
                pos = (lax.iota(jnp.int32, LANES) + g * LANES) * L + l
                vals = plsc.load_gather(idx_v, [pos])
                idxt_v[l, pl.ds(g * LANES, LANES)] = vals

        def gather(l, buf):
            return pltpu.make_async_copy(
                table_hbm.at[idxt_v.at[l]],
                acc_v.at[buf],
                sems[buf],
            )

        gather(0, 0).start()
        gather(1, 1).start()

        @pl.loop(2, L, step=2)
        def _gathers(l):
            gather(l, 0).wait()
            gather(l, 0).start(add=True)
            gather(l + 1, 1).wait()
            gather(l + 1, 1).start(add=True)

        gather(L - 2, 0).wait()
        gather(L - 1, 1).wait()

        @pl.loop(0, b_per_w)
        def _fold(i):
            for j in range(NVREG):
                s = (acc_v[0, i, pl.ds(j * LANES, LANES)]
                     + acc_v[1, i, pl.ds(j * LANES, LANES)])
                acc_v[0, i, pl.ds(j * LANES, LANES)] = s * (1.0 / L)

        pltpu.sync_copy(acc_v.at[0], out_hbm.at[pl.ds(base_row, b_per_w)])

    return k


def kernel(text_ids, table):
    return _build()(text_ids.reshape(-1).astype(jnp.int32), table)


# speedup vs baseline: 1.1101x; 1.1101x over previous
import functools

import jax
import jax.numpy as jnp
from jax import lax
from jax.experimental import pallas as pl
from jax.experimental.pallas import tpu as pltpu
from jax.experimental.pallas import tpu_sc as plsc

B = 4096
L = 50
D = 128
LANES = 16
NVREG = D // LANES
NBUF = 5


@functools.cache
def _build():
    info = plsc.get_sparse_core_info()
    nw = info.num_cores * info.num_subcores
    b_per_w = B // nw
    assert L % NBUF == 0
    mesh = plsc.VectorSubcoreMesh(core_axis_name="c", subcore_axis_name="s")

    @functools.partial(
        pl.kernel,
        mesh=mesh,
        compiler_params=pltpu.CompilerParams(needs_layout_passes=False),
        out_type=jax.ShapeDtypeStruct((B, D), jnp.float32),
        scratch_types=[
            pltpu.VMEM((b_per_w * L,), jnp.int32),
            pltpu.VMEM((L, b_per_w), jnp.int32),
            pltpu.VMEM((NBUF, b_per_w, D), jnp.float32),
            [pltpu.SemaphoreType.DMA] * NBUF,
        ],
    )
    def k(ids_hbm, table_hbm, out_hbm, idx_v, idxt_v, acc_v, sems):
        cid = lax.axis_index("c")
        sid = lax.axis_index("s")
        wid = sid * info.num_cores + cid
        base_row = wid * b_per_w

        pltpu.sync_copy(ids_hbm.at[pl.ds(base_row * L, b_per_w * L)], idx_v)

        @pl.loop(0, L)
        def _transpose(l):
            for g in range(b_per_w // LANES):
                pos = (lax.iota(jnp.int32, LANES) + g * LANES) * L + l
                vals = plsc.load_gather(idx_v, [pos])
                idxt_v[l, pl.ds(g * LANES, LANES)] = vals

        def gather(l, buf):
            return pltpu.make_async_copy(
                table_hbm.at[idxt_v.at[l]],
                acc_v.at[buf],
                sems[buf],
            )

        for n in range(NBUF):
            gather(n, n).start()

        @pl.loop(NBUF, L, step=NBUF)
        def _gathers(l):
            for n in range(NBUF):
                gather(l + n, n).wait()
                gather(l + n, n).start(add=True)

        for n in range(NBUF):
            gather(n, n).wait()

        @pl.loop(0, b_per_w)
        def _fold(i):
            for j in range(NVREG):
                s = acc_v[0, i, pl.ds(j * LANES, LANES)]
                for n in range(1, NBUF):
                    s = s + acc_v[n, i, pl.ds(j * LANES, LANES)]
                acc_v[0, i, pl.ds(j * LANES, LANES)] = s * (1.0 / L)

        pltpu.sync_copy(acc_v.at[0], out_hbm.at[pl.ds(base_row, b_per_w)])

    return k


def kernel(text_ids, table):
    return _build()(text_ids.reshape(-1).astype(jnp.int32), table)
